# Initial kernel scaffold; baseline (speedup 1.0000x reference)
import jax
import jax.numpy as jnp
from jax.experimental import pallas as pl


def kernel(imgs, anchors, regression, classification):
    raise NotImplementedError("write your pallas kernel here")


# speedup vs baseline: 9.0392x; 9.0392x over previous
import functools

import jax
import jax.numpy as jnp
from jax.experimental import pallas as pl

N = 20000
NPAD = 20480
ROWS = 160
LANES = 128
NCLS = 90
KDET = 100
NEG = -1e9
SCORE_THRESH = 0.05


def _nms_body(anchors_ref, regression_ref, cls_ref, out_ref, *, height, width):
    a = anchors_ref[...]
    ya1, xa1, ya2, xa2 = a[0], a[1], a[2], a[3]
    r = regression_ref[0]
    dy, dx, dh, dw = r[0], r[1], r[2], r[3]

    cya = (ya1 + ya2) * 0.5
    cxa = (xa1 + xa2) * 0.5
    ha = ya2 - ya1
    wa = xa2 - xa1
    w = jnp.exp(dw) * wa
    h = jnp.exp(dh) * ha
    yc = dy * ha + cya
    xc = dx * wa + cxa
    x1 = jnp.clip(xc - w * 0.5, 0.0, width)
    y1 = jnp.clip(yc - h * 0.5, 0.0, height)
    x2 = jnp.clip(xc + w * 0.5, 0.0, width)
    y2 = jnp.clip(yc + h * 0.5, 0.0, height)

    c = cls_ref[0]
    sc = jnp.max(c, axis=0)
    cit = jax.lax.broadcasted_iota(jnp.int32, (NCLS, ROWS, LANES), 0)
    cls_i = jnp.min(jnp.where(c == sc[None], cit, NCLS), axis=0)
    clsf = cls_i.astype(jnp.float32)

    rowi = jax.lax.broadcasted_iota(jnp.int32, (ROWS, LANES), 0)
    coli = jax.lax.broadcasted_iota(jnp.int32, (ROWS, LANES), 1)
    flat = rowi * LANES + coli
    s0 = jnp.where((flat < N) & (sc > SCORE_THRESH), sc, NEG)

    max_coord = max(height, width) + 1.0
    off = clsf * max_coord
    x1n = x1 + off
    y1n = y1 + off
    x2n = x2 + off
    y2n = y2 + off
    areas = jnp.maximum(x2n - x1n, 0.0) * jnp.maximum(y2n - y1n, 0.0)

    lane = jax.lax.broadcasted_iota(jnp.int32, (1, LANES), 1)
    zlane = jnp.zeros((1, LANES), jnp.float32)

    def body(i, carry):
        s, o1, o2, o3, o4, o5, o6 = carry
        m = jnp.max(s)
        valid = m > NEG * 0.5
        bi = jnp.min(jnp.where(s == m, flat, NPAD))
        mb = flat == bi
        xb1 = jnp.sum(jnp.where(mb, x1n, 0.0))
        yb1 = jnp.sum(jnp.where(mb, y1n, 0.0))
        xb2 = jnp.sum(jnp.where(mb, x2n, 0.0))
        yb2 = jnp.sum(jnp.where(mb, y2n, 0.0))
        cb = jnp.sum(jnp.where(mb, clsf, 0.0))
        area_b = jnp.maximum(xb2 - xb1, 0.0) * jnp.maximum(yb2 - yb1, 0.0)
        iw = jnp.maximum(jnp.minimum(xb2, x2n) - jnp.maximum(xb1, x1n), 0.0)
        ih = jnp.maximum(jnp.minimum(yb2, y2n) - jnp.maximum(yb1, y1n), 0.0)
        inter = iw * ih
        denom = area_b + areas - inter + 1e-8
        supp = (inter > 0.5 * denom) & valid
        s = jnp.where(supp | mb, NEG, s)
        offb = cb * max_coord
        hit = (lane == i) & valid
        o1 = jnp.where(hit, xb1 - offb, o1)
        o2 = jnp.where(hit, yb1 - offb, o2)
        o3 = jnp.where(hit, xb2 - offb, o3)
        o4 = jnp.where(hit, yb2 - offb, o4)
        o5 = jnp.where(hit, m, o5)
        o6 = jnp.where(hit, cb + 1.0, o6)
        return (s, o1, o2, o3, o4, o5, o6)

    init = (s0, zlane, zlane, zlane, zlane, zlane, zlane)
    _, o1, o2, o3, o4, o5, o6 = jax.lax.fori_loop(0, KDET, body, init)
    out_ref[0] = jnp.concatenate([o1, o2, o3, o4, o5, o6, zlane, zlane], axis=0)


def kernel(imgs, anchors, regression, classification):
    height = float(imgs.shape[2])
    width = float(imgs.shape[3])
    B = regression.shape[0]

    at = jnp.transpose(anchors[0], (1, 0))
    at = jnp.pad(at, ((0, 0), (0, NPAD - N))).reshape(4, ROWS, LANES)
    rt = jnp.transpose(regression, (0, 2, 1))
    rt = jnp.pad(rt, ((0, 0), (0, 0), (0, NPAD - N))).reshape(B, 4, ROWS, LANES)
    ct = jnp.transpose(classification, (0, 2, 1))
    ct = jnp.pad(ct, ((0, 0), (0, 0), (0, NPAD - N)),
                 constant_values=-1.0).reshape(B, NCLS, ROWS, LANES)

    out_planes = pl.pallas_call(
        functools.partial(_nms_body, height=height, width=width),
        grid=(B,),
        in_specs=[
            pl.BlockSpec((4, ROWS, LANES), lambda b: (0, 0, 0)),
            pl.BlockSpec((1, 4, ROWS, LANES), lambda b: (b, 0, 0, 0)),
            pl.BlockSpec((1, NCLS, ROWS, LANES), lambda b: (b, 0, 0, 0)),
        ],
        out_specs=pl.BlockSpec((1, 8, LANES), lambda b: (b, 0, 0)),
        out_shape=jax.ShapeDtypeStruct((B, 8, LANES), jnp.float32),
    )(at, rt, ct)

    return jnp.transpose(out_planes[:, :6, :KDET], (0, 2, 1))
